# Initial kernel scaffold; baseline (speedup 1.0000x reference)
import jax
import jax.numpy as jnp
from jax.experimental import pallas as pl


def kernel(x, idx):
    raise NotImplementedError("write your pallas kernel here")


# speedup vs baseline: 17.8735x; 17.8735x over previous
import functools

import jax
import jax.numpy as jnp
from jax import lax
from jax.experimental import pallas as pl
from jax.experimental.pallas import tpu as pltpu
from jax.experimental.pallas import tpu_sc as plsc


def kernel(x, idx):
    b, t, _ = x.shape
    num = idx.shape[1]
    info = plsc.get_sparse_core_info()
    lanes = info.num_lanes
    nw = info.num_cores * info.num_subcores
    rows_per_w = b // nw

    mesh = plsc.VectorSubcoreMesh(core_axis_name="c", subcore_axis_name="s")

    @functools.partial(
        pl.kernel,
        out_type=jax.ShapeDtypeStruct((b, t), jnp.float32),
        mesh=mesh,
        compiler_params=pltpu.CompilerParams(needs_layout_passes=False),
        scratch_types=[
            pltpu.VMEM((t,), jnp.float32),
            pltpu.VMEM((num,), jnp.int32),
            pltpu.SemaphoreType.DMA,
            pltpu.SemaphoreType.DMA,
        ],
    )
    def cut(x_hbm, idx_hbm, out_hbm, row_v, idx_v, sem_x, sem_i):
        wid = lax.axis_index("s") * info.num_cores + lax.axis_index("c")
        zeros = jnp.zeros((lanes,), jnp.float32)

        def per_row(i, carry):
            r = wid * rows_per_w + i
            cx = pltpu.async_copy(x_hbm.at[r], row_v, sem_x)
            ci = pltpu.async_copy(idx_hbm.at[r], idx_v, sem_i)
            ci.wait()
            cx.wait()

            def scat(j, c):
                v = idx_v[pl.ds(j * lanes, lanes)]
                plsc.store_scatter(row_v, [v], zeros)
                return c

            lax.fori_loop(0, num // lanes, scat, 0)
            pltpu.sync_copy(row_v, out_hbm.at[r])
            return carry

        lax.fori_loop(0, rows_per_w, per_row, 0)

    out = cut(jnp.reshape(x, (b, t)), idx)
    return jnp.reshape(out, (b, t, 1))


# speedup vs baseline: 19.6580x; 1.0998x over previous
import functools

import jax
import jax.numpy as jnp
from jax import lax
from jax.experimental import pallas as pl
from jax.experimental.pallas import tpu as pltpu
from jax.experimental.pallas import tpu_sc as plsc


def kernel(x, idx):
    b, t, _ = x.shape
    num = idx.shape[1]
    info = plsc.get_sparse_core_info()
    lanes = info.num_lanes
    nw = info.num_cores * info.num_subcores
    rows_per_w = b // nw

    mesh = plsc.VectorSubcoreMesh(core_axis_name="c", subcore_axis_name="s")

    @functools.partial(
        pl.kernel,
        out_type=jax.ShapeDtypeStruct((b, t), jnp.float32),
        mesh=mesh,
        compiler_params=pltpu.CompilerParams(needs_layout_passes=False),
        scratch_types=[
            pltpu.VMEM((t,), jnp.float32),
            pltpu.VMEM((t,), jnp.float32),
            pltpu.VMEM((rows_per_w * num,), jnp.int32),
            pltpu.SemaphoreType.DMA,
            pltpu.SemaphoreType.DMA,
            pltpu.SemaphoreType.DMA,
            pltpu.SemaphoreType.DMA,
            pltpu.SemaphoreType.DMA,
        ],
    )
    def cut(x_hbm, idx_hbm, out_hbm, row_v0, row_v1, idx_v, semi, semx0, semx1, semo0, semo1):
        wid = lax.axis_index("s") * info.num_cores + lax.axis_index("c")
        base = wid * rows_per_w
        zeros = jnp.zeros((lanes,), jnp.float32)
        bufs = (row_v0, row_v1)
        semx = (semx0, semx1)
        semo = (semo0, semo1)

        ci = pltpu.async_copy(idx_hbm.at[pl.ds(base * num, rows_per_w * num)], idx_v, semi)
        loads = [None] * rows_per_w
        stores = [None] * rows_per_w

        def start_load(i):
            loads[i] = pltpu.async_copy(x_hbm.at[base + i], bufs[i % 2], semx[i % 2])

        start_load(0)
        ci.wait()
        for i in range(rows_per_w):
            if i + 1 < rows_per_w:
                if i >= 1:
                    stores[i - 1].wait()
                start_load(i + 1)
            loads[i].wait()

            def scat(j, c, i=i):
                v = idx_v[pl.ds(i * num + j * lanes, lanes)]
                plsc.store_scatter(bufs[i % 2], [v], zeros)
                return c

            lax.fori_loop(0, num // lanes, scat, 0)
            stores[i] = pltpu.async_copy(bufs[i % 2], out_hbm.at[base + i], semo[i % 2])
        stores[-2].wait()
        stores[-1].wait()

    out = cut(jnp.reshape(x, (b, t)), jnp.reshape(idx, (b * num,)))
    return jnp.reshape(out, (b, t, 1))


# speedup vs baseline: 46.0490x; 2.3425x over previous
import functools

import jax
import jax.numpy as jnp
from jax import lax
from jax.experimental import pallas as pl
from jax.experimental.pallas import tpu as pltpu
from jax.experimental.pallas import tpu_sc as plsc


def kernel(x, idx):
    b, t, _ = x.shape
    num = idx.shape[1]
    info = plsc.get_sparse_core_info()
    lanes = info.num_lanes
    nw = info.num_cores * info.num_subcores
    rows_per_w = b // nw

    mesh = plsc.VectorSubcoreMesh(core_axis_name="c", subcore_axis_name="s")

    @functools.partial(
        pl.kernel,
        out_type=jax.ShapeDtypeStruct((b * t,), jnp.float32),
        mesh=mesh,
        compiler_params=pltpu.CompilerParams(needs_layout_passes=False),
        scratch_types=[
            pltpu.VMEM((t,), jnp.float32),
            pltpu.VMEM((t,), jnp.float32),
            pltpu.VMEM((t,), jnp.float32),
            pltpu.VMEM((rows_per_w, num), jnp.int32),
            pltpu.SemaphoreType.DMA,
            pltpu.SemaphoreType.DMA,
            pltpu.SemaphoreType.DMA,
            pltpu.SemaphoreType.DMA,
            pltpu.SemaphoreType.DMA,
            pltpu.SemaphoreType.DMA,
            pltpu.SemaphoreType.DMA,
        ],
    )
    def cut(x_hbm, idx_hbm, out_hbm, row_v0, row_v1, row_v2, idx_v,
            semi, semx0, semx1, semx2, semo0, semo1, semo2):
        wid = lax.axis_index("s") * info.num_cores + lax.axis_index("c")
        base = wid * rows_per_w
        zeros = jnp.zeros((lanes,), jnp.float32)
        bufs = (row_v0, row_v1, row_v2)
        semx = (semx0, semx1, semx2)
        semo = (semo0, semo1, semo2)
        nbuf = 3

        ci = pltpu.async_copy(idx_hbm.at[pl.ds(base, rows_per_w)], idx_v, semi)
        loads = [None] * rows_per_w
        stores = [None] * rows_per_w

        h = t // 2

        def start_load(i):
            loads[i] = (
                pltpu.async_copy(
                    x_hbm.at[pl.ds((base + i) * t, h)],
                    bufs[i % nbuf].at[pl.ds(0, h)], semx[i % nbuf]),
                pltpu.async_copy(
                    x_hbm.at[pl.ds((base + i) * t + h, h)],
                    bufs[i % nbuf].at[pl.ds(h, h)], semx[i % nbuf]),
            )

        start_load(0)
        start_load(1)
        ci.wait()
        for i in range(rows_per_w):
            if i + 2 < rows_per_w:
                if i >= 1:
                    for s in stores[i - 1]:
                        s.wait()
                start_load(i + 2)
            for l in loads[i]:
                l.wait()

            @plsc.parallel_loop(0, num, step=lanes, unroll=8)
            def _(j, i=i):
                v = idx_v[i, pl.ds(j, lanes)]
                plsc.store_scatter(bufs[i % nbuf], [v], zeros)
            stores[i] = (
                pltpu.async_copy(
                    bufs[i % nbuf].at[pl.ds(0, h)],
                    out_hbm.at[pl.ds((base + i) * t, h)], semo[i % nbuf]),
                pltpu.async_copy(
                    bufs[i % nbuf].at[pl.ds(h, h)],
                    out_hbm.at[pl.ds((base + i) * t + h, h)], semo[i % nbuf]),
            )
        for pair in stores[-3:]:
            for s in pair:
                s.wait()

    out = cut(jnp.reshape(x, (b * t,)), idx)
    return jnp.reshape(out, (b, t, 1))
